# baseline (speedup 1.0000x reference)
import jax
import jax.numpy as jnp
from jax.experimental import pallas as pl
from jax.experimental.pallas import tpu as pltpu

_BS = 832


def _ln_add_block(x_ref, pos_ref, o_ref):
    x = x_ref[...]
    mean = jnp.mean(x, axis=-1, keepdims=True)
    xc = x - mean
    var = jnp.mean(xc * xc, axis=-1, keepdims=True)
    o_ref[...] = xc * jax.lax.rsqrt(var + 1e-5) + pos_ref[...]


def kernel(token_embeddings, pos_table):
    b, s, d = token_embeddings.shape
    grid = (pl.cdiv(s, _BS),)
    return pl.pallas_call(
        _ln_add_block,
        grid=grid,
        in_specs=[
            pl.BlockSpec((b, _BS, d), lambda i: (0, i, 0)),
            pl.BlockSpec((_BS, d), lambda i: (i, 0)),
        ],
        out_specs=pl.BlockSpec((b, _BS, d), lambda i: (0, i, 0)),
        out_shape=jax.ShapeDtypeStruct((b, s, d), token_embeddings.dtype),
        compiler_params=pltpu.CompilerParams(
            dimension_semantics=("arbitrary",),
            vmem_limit_bytes=67108864,
        ),
    )(token_embeddings, pos_table[:s])


# speedup vs baseline: 1.0126x; 1.0126x over previous
import jax
import jax.numpy as jnp
from jax import lax
from jax.experimental import pallas as pl
from jax.experimental.pallas import tpu as pltpu

_CH = 512
_NBUF = 4


def _ln_add_manual(tok_hbm, pos_hbm, out_hbm, inb, outb, posb,
                   in_sem, out_sem, pos_sem):
    bsz_x_seq, d = tok_hbm.shape
    seq = pos_hbm.shape[0]
    bsz = bsz_x_seq // seq
    n_chunks = seq // _CH
    n_iter = n_chunks * bsz

    def tok_in(i, slot):
        c, b = i // bsz, i % bsz
        return pltpu.make_async_copy(
            tok_hbm.at[pl.ds(b * seq + c * _CH, _CH)], inb.at[slot],
            in_sem.at[slot])

    def pos_in(c):
        return pltpu.make_async_copy(
            pos_hbm.at[pl.ds(c * _CH, _CH)], posb.at[c % 2], pos_sem.at[c % 2])

    def res_out(i, slot):
        c, b = i // bsz, i % bsz
        return pltpu.make_async_copy(
            outb.at[slot], out_hbm.at[pl.ds(b * seq + c * _CH, _CH)],
            out_sem.at[slot])

    pos_in(0).start()
    for i in range(_NBUF - 1):
        tok_in(i, i).start()
        if i % bsz == bsz - 1 and (i + 1) // bsz < n_chunks:
            pos_in((i + 1) // bsz).start()

    def body(i, carry):
        slot = i % _NBUF
        pre = i + _NBUF - 1

        @pl.when(pre < n_iter)
        def _():
            tok_in(pre, pre % _NBUF).start()

        @pl.when((pre % bsz == bsz - 1) & (pre + 1 < n_iter))
        def _():
            pos_in((pre + 1) // bsz).start()

        tok_in(i, slot).wait()

        @pl.when(i % bsz == 0)
        def _():
            pos_in(i // bsz).wait()

        @pl.when(i >= _NBUF)
        def _():
            res_out(i - _NBUF, slot).wait()

        x = inb[slot]
        mean = jnp.mean(x, axis=-1, keepdims=True)
        xc = x - mean
        var = jnp.mean(xc * xc, axis=-1, keepdims=True)
        outb[slot] = xc * lax.rsqrt(var + 1e-5) + posb[(i // bsz) % 2]
        res_out(i, slot).start()
        return carry

    lax.fori_loop(0, n_iter, body, 0)
    for k in range(_NBUF):
        res_out(n_iter - _NBUF + k, (n_iter - _NBUF + k) % _NBUF).wait()


def kernel(token_embeddings, pos_table):
    b, s, d = token_embeddings.shape
    out_flat = pl.pallas_call(
        _ln_add_manual,
        in_specs=[
            pl.BlockSpec(memory_space=pl.ANY),
            pl.BlockSpec(memory_space=pl.ANY),
        ],
        out_specs=pl.BlockSpec(memory_space=pl.ANY),
        out_shape=jax.ShapeDtypeStruct((b * s, d), token_embeddings.dtype),
        scratch_shapes=[
            pltpu.VMEM((_NBUF, _CH, 1024), jnp.float32),
            pltpu.VMEM((_NBUF, _CH, 1024), jnp.float32),
            pltpu.VMEM((2, _CH, 1024), jnp.float32),
            pltpu.SemaphoreType.DMA((_NBUF,)),
            pltpu.SemaphoreType.DMA((_NBUF,)),
            pltpu.SemaphoreType.DMA((2,)),
        ],
    )(token_embeddings.reshape(b * s, d), pos_table[:s])
    return out_flat.reshape(b, s, d)


# speedup vs baseline: 1.0162x; 1.0036x over previous
import jax
import jax.numpy as jnp
from jax import lax
from jax.experimental import pallas as pl
from jax.experimental.pallas import tpu as pltpu

_CH = 1024
_NBUF = 4


def _ln_add_manual(tok_hbm, pos_hbm, out_hbm, inb, outb, posb,
                   in_sem, out_sem, pos_sem):
    bsz_x_seq, d = tok_hbm.shape
    seq = pos_hbm.shape[0]
    bsz = bsz_x_seq // seq
    n_chunks = seq // _CH
    n_iter = n_chunks * bsz

    def tok_in(i, slot):
        c, b = i // bsz, i % bsz
        return pltpu.make_async_copy(
            tok_hbm.at[pl.ds(b * seq + c * _CH, _CH)], inb.at[slot],
            in_sem.at[slot])

    def pos_in(c):
        return pltpu.make_async_copy(
            pos_hbm.at[pl.ds(c * _CH, _CH)], posb.at[c % 2], pos_sem.at[c % 2])

    def res_out(i, slot):
        c, b = i // bsz, i % bsz
        return pltpu.make_async_copy(
            outb.at[slot], out_hbm.at[pl.ds(b * seq + c * _CH, _CH)],
            out_sem.at[slot])

    pos_in(0).start()
    for i in range(_NBUF - 1):
        tok_in(i, i).start()
        if i % bsz == bsz - 1 and (i + 1) // bsz < n_chunks:
            pos_in((i + 1) // bsz).start()

    def body(i, carry):
        slot = i % _NBUF
        pre = i + _NBUF - 1

        @pl.when(pre < n_iter)
        def _():
            tok_in(pre, pre % _NBUF).start()

        @pl.when((pre % bsz == bsz - 1) & (pre + 1 < n_iter))
        def _():
            pos_in((pre + 1) // bsz).start()

        tok_in(i, slot).wait()

        @pl.when(i % bsz == 0)
        def _():
            pos_in(i // bsz).wait()

        @pl.when(i >= _NBUF)
        def _():
            res_out(i - _NBUF, slot).wait()

        x = inb[slot]
        mean = jnp.mean(x, axis=-1, keepdims=True)
        xc = x - mean
        var = jnp.mean(xc * xc, axis=-1, keepdims=True)
        outb[slot] = xc * lax.rsqrt(var + 1e-5) + posb[(i // bsz) % 2]
        res_out(i, slot).start()
        return carry

    lax.fori_loop(0, n_iter, body, 0)
    for k in range(_NBUF):
        res_out(n_iter - _NBUF + k, (n_iter - _NBUF + k) % _NBUF).wait()


def kernel(token_embeddings, pos_table):
    b, s, d = token_embeddings.shape
    out_flat = pl.pallas_call(
        _ln_add_manual,
        in_specs=[
            pl.BlockSpec(memory_space=pl.ANY),
            pl.BlockSpec(memory_space=pl.ANY),
        ],
        out_specs=pl.BlockSpec(memory_space=pl.ANY),
        out_shape=jax.ShapeDtypeStruct((b * s, d), token_embeddings.dtype),
        scratch_shapes=[
            pltpu.VMEM((_NBUF, _CH, 1024), jnp.float32),
            pltpu.VMEM((_NBUF, _CH, 1024), jnp.float32),
            pltpu.VMEM((2, _CH, 1024), jnp.float32),
            pltpu.SemaphoreType.DMA((_NBUF,)),
            pltpu.SemaphoreType.DMA((_NBUF,)),
            pltpu.SemaphoreType.DMA((2,)),
        ],
    )(token_embeddings.reshape(b * s, d), pos_table[:s])
    return out_flat.reshape(b, s, d)
